# baseline (speedup 1.0000x reference)
import functools

import jax
import jax.numpy as jnp
from jax import lax
from jax.experimental import pallas as pl
from jax.experimental.pallas import tpu as pltpu
from jax.experimental.pallas import tpu_sc as plsc

N = 2048
E = 65536
G = 8
ALPHA = 0.15
K_TOP = 64 * N

_BM = 1024
_BN = 1024
_RB = 256


_CHROWS = 512
_CHELEMS = _CHROWS * N
_SPAD = 2048
_ZSLICE = (_CHELEMS + _SPAD) // 16
_EPW = E // 16


def _sc_scatter_body(src_hbm, dst_hbm, zero_hbm, a_hbm, sh, s_v, d_v, i_v,
                     v_v):
    cid = lax.axis_index("c")
    sid = lax.axis_index("s")

    ones = jnp.full((16,), 1.0, jnp.float32)

    def fill(i, _):
        v_v[pl.ds(i * 16, 16)] = ones
        return 0

    lax.fori_loop(0, _EPW // 16, fill, 0)

    ebase = sid * _EPW
    pltpu.sync_copy(src_hbm.at[pl.ds(ebase, _EPW)], s_v)
    pltpu.sync_copy(dst_hbm.at[pl.ds(ebase, _EPW)], d_v)

    for ch in range(2):
        rbase = (cid * 2 + ch) * _CHROWS
        pltpu.sync_copy(zero_hbm.at[pl.ds(sid * _ZSLICE, _ZSLICE)],
                        sh.at[pl.ds(sid * _ZSLICE, _ZSLICE)])
        plsc.subcore_barrier()

        def idx_body(i, _):
            s16 = s_v[pl.ds(i * 16, 16)]
            d16 = d_v[pl.ds(i * 16, 16)]
            m = jnp.logical_and(s16 >= rbase, s16 < rbase + _CHROWS)
            flat = (s16 - rbase) * N + d16
            i_v[pl.ds(i * 16, 16)] = jnp.where(m, flat, _CHELEMS)
            return 0

        lax.fori_loop(0, _EPW // 16, idx_body, 0)
        for t in range(16):
            @pl.when(sid == t)
            def _():
                pltpu.sync_copy(v_v, sh.at[i_v], add=True)

            plsc.subcore_barrier()

        osz = _CHELEMS // 16
        pltpu.sync_copy(sh.at[pl.ds(sid * osz, osz)],
                        a_hbm.at[pl.ds(rbase * N + sid * osz, osz)])
        plsc.subcore_barrier()


def _sc_scatter(src, dst, zero_blk):
    return pl.kernel(
        _sc_scatter_body,
        out_type=jax.ShapeDtypeStruct((N * N,), jnp.float32),
        mesh=plsc.VectorSubcoreMesh(core_axis_name="c", subcore_axis_name="s"),
        compiler_params=pltpu.CompilerParams(needs_layout_passes=False),
        scratch_types=[
            pltpu.VMEM_SHARED((_CHELEMS + _SPAD,), jnp.float32),
            pltpu.VMEM((_EPW,), jnp.int32),
            pltpu.VMEM((_EPW,), jnp.int32),
            pltpu.VMEM((_EPW,), jnp.int32),
            pltpu.VMEM((_EPW,), jnp.float32),
        ],
    )(src, dst, zero_blk)


def _colsum_body(a_ref, o_ref):
    i = pl.program_id(0)

    @pl.when(i == 0)
    def _():
        o_ref[...] = jnp.zeros_like(o_ref)

    o_ref[...] += jnp.sum(a_ref[...], axis=0, keepdims=True)


def _colsum(a):
    return pl.pallas_call(
        _colsum_body,
        grid=(N // _RB,),
        in_specs=[pl.BlockSpec((_RB, N), lambda i: (i, 0))],
        out_specs=pl.BlockSpec((1, N), lambda i: (0, 0)),
        out_shape=jax.ShapeDtypeStruct((1, N), jnp.float32),
    )(a)


def _norm_body(a_ref, deg_ref, degt_ref, b_ref, p_ref):
    i = pl.program_id(0)
    rows = lax.broadcasted_iota(jnp.int32, (_RB, N), 0) + i * _RB
    cols = lax.broadcasted_iota(jnp.int32, (_RB, N), 1)
    eye = (rows == cols).astype(jnp.float32)
    deg = deg_ref[...]
    degt = degt_ref[...]
    dinv_c = jnp.where(deg > 0, lax.rsqrt(deg), 0.0)
    dinv_r = jnp.where(degt > 0, lax.rsqrt(degt), 0.0)
    b = (1.0 - ALPHA) * ((a_ref[...] + eye) * dinv_r * dinv_c)
    b_ref[...] = b
    p_ref[...] = b + eye


def _normalize(a, deg, deg_t):
    return pl.pallas_call(
        _norm_body,
        grid=(N // _RB,),
        in_specs=[
            pl.BlockSpec((_RB, N), lambda i: (i, 0)),
            pl.BlockSpec((1, N), lambda i: (0, 0)),
            pl.BlockSpec((_RB, 1), lambda i: (i, 0)),
        ],
        out_specs=[
            pl.BlockSpec((_RB, N), lambda i: (i, 0)),
            pl.BlockSpec((_RB, N), lambda i: (i, 0)),
        ],
        out_shape=[
            jax.ShapeDtypeStruct((N, N), jnp.float32),
            jax.ShapeDtypeStruct((N, N), jnp.float32),
        ],
    )(a, deg, deg_t)


def _mm_body(x_ref, y_ref, o_ref):
    o_ref[...] = jnp.dot(x_ref[...], y_ref[...],
                         preferred_element_type=jnp.float32)


def _mma_body(x_ref, y_ref, c_ref, o_ref):
    o_ref[...] = jnp.dot(x_ref[...], y_ref[...],
                         preferred_element_type=jnp.float32) + c_ref[...]


def _mm(x, y):
    return pl.pallas_call(
        _mm_body,
        grid=(N // _BM, N // _BN),
        in_specs=[
            pl.BlockSpec((_BM, N), lambda i, j: (i, 0)),
            pl.BlockSpec((N, _BN), lambda i, j: (0, j)),
        ],
        out_specs=pl.BlockSpec((_BM, _BN), lambda i, j: (i, j)),
        out_shape=jax.ShapeDtypeStruct((N, N), jnp.float32),
        compiler_params=pltpu.CompilerParams(
            dimension_semantics=("parallel", "parallel"),
            vmem_limit_bytes=100 * 1024 * 1024),
    )(x, y)


def _mma(x, y, c):
    return pl.pallas_call(
        _mma_body,
        grid=(N // _BM, N // _BN),
        in_specs=[
            pl.BlockSpec((_BM, N), lambda i, j: (i, 0)),
            pl.BlockSpec((N, _BN), lambda i, j: (0, j)),
            pl.BlockSpec((_BM, _BN), lambda i, j: (i, j)),
        ],
        out_specs=pl.BlockSpec((_BM, _BN), lambda i, j: (i, j)),
        out_shape=jax.ShapeDtypeStruct((N, N), jnp.float32),
        compiler_params=pltpu.CompilerParams(
            dimension_semantics=("parallel", "parallel"),
            vmem_limit_bytes=100 * 1024 * 1024),
    )(x, y, c)


_N_ITERS = 31
_POS_INF_BITS = 0x7F800000


def _select_body(p_ref, eps_ref, dg_ref):
    nblk = N // _RB

    def count_ge(mid1, mid2):
        c1 = jnp.int32(0)
        c2 = jnp.int32(0)
        for b in range(nblk):
            bits = lax.bitcast_convert_type(
                p_ref[pl.ds(b * _RB, _RB), :], jnp.int32)
            c1 += jnp.sum((bits >= mid1).astype(jnp.int32))
            c2 += jnp.sum((bits >= mid2).astype(jnp.int32))
        return c1, c2

    def body(_, carry):
        lo1, hi1, lo2, hi2 = carry
        mid1 = lo1 + (hi1 - lo1) // 2
        mid2 = lo2 + (hi2 - lo2) // 2
        c1, c2 = count_ge(mid1, mid2)
        ge1 = c1 >= K_TOP
        ge2 = c2 >= (K_TOP + 1)
        return (jnp.where(ge1, mid1, lo1), jnp.where(ge1, hi1, mid1),
                jnp.where(ge2, mid2, lo2), jnp.where(ge2, hi2, mid2))

    init = (jnp.int32(0), jnp.int32(_POS_INF_BITS),
            jnp.int32(0), jnp.int32(_POS_INF_BITS))
    lo1, _, lo2, _ = lax.fori_loop(0, _N_ITERS, body, init)
    vk = lax.bitcast_convert_type(lo1, jnp.float32)
    vk1 = lax.bitcast_convert_type(lo2, jnp.float32)
    eps = (vk + vk1) * 0.5
    eps_ref[0] = eps

    dg = jnp.zeros((1, N), jnp.float32)
    for b in range(nblk):
        rows = lax.broadcasted_iota(jnp.int32, (_RB, N), 0) + b * _RB
        cols = lax.broadcasted_iota(jnp.int32, (_RB, N), 1)
        adjf = (p_ref[pl.ds(b * _RB, _RB), :] >= eps).astype(jnp.float32)
        ag = jnp.where(rows == cols, 1.0, adjf)
        dg += jnp.sum(ag, axis=0, keepdims=True)
    dg_ref[...] = dg


def _select_eps(p):
    return pl.pallas_call(
        _select_body,
        out_specs=[
            pl.BlockSpec(memory_space=pltpu.SMEM),
            pl.BlockSpec((1, N), lambda: (0, 0)),
        ],
        out_shape=[
            jax.ShapeDtypeStruct((1,), jnp.float32),
            jax.ShapeDtypeStruct((1, N), jnp.float32),
        ],
        compiler_params=pltpu.CompilerParams(
            vmem_limit_bytes=50 * 1024 * 1024),
    )(p)


def _gat_body(p_ref, eps_ref, x_ref, wg_ref, asrc_ref, adst_ref, bg_ref,
              x1_ref):
    i = pl.program_id(0)
    eps = eps_ref[0]
    h = x_ref[...] * wg_ref[...]
    a_s = jnp.dot(h, asrc_ref[...], preferred_element_type=jnp.float32)
    h_blk = x_ref[pl.ds(i * _RB, _RB), :] * wg_ref[...]
    a_d = lax.dot_general(adst_ref[...], h_blk,
                          dimension_numbers=(((0,), (1,)), ((), ())),
                          preferred_element_type=jnp.float32)
    rows = lax.broadcasted_iota(jnp.int32, (N, _RB), 0)
    cols = lax.broadcasted_iota(jnp.int32, (N, _RB), 1) + i * _RB
    mask = jnp.logical_or(p_ref[...] >= eps, rows == cols)
    e = a_s + a_d
    e = jnp.where(e >= 0, e, 0.2 * e)
    e = jnp.where(mask, e, -1e9)
    m = jnp.max(e, axis=0, keepdims=True)
    pexp = jnp.exp(e - m)
    z = jnp.sum(pexp, axis=0, keepdims=True)
    attn = jnp.where(mask, pexp / z, 0.0)
    v = lax.dot_general(attn, h, dimension_numbers=(((0,), (0,)), ((), ())),
                        preferred_element_type=jnp.float32) + bg_ref[...]
    x1_ref[...] = jnp.where(v > 0, v, jnp.exp(v) - 1.0)


def _gat(p, eps, x, w_gat, att_src, att_dst, b_gat):
    return pl.pallas_call(
        _gat_body,
        grid=(N // _RB,),
        in_specs=[
            pl.BlockSpec((N, _RB), lambda i: (0, i)),
            pl.BlockSpec(memory_space=pltpu.SMEM),
            pl.BlockSpec((N, 1), lambda i: (0, 0)),
            pl.BlockSpec((1, 16), lambda i: (0, 0)),
            pl.BlockSpec((16, 1), lambda i: (0, 0)),
            pl.BlockSpec((16, 1), lambda i: (0, 0)),
            pl.BlockSpec((1, 16), lambda i: (0, 0)),
        ],
        out_specs=pl.BlockSpec((_RB, 16), lambda i: (i, 0)),
        out_shape=jax.ShapeDtypeStruct((N, 16), jnp.float32),
    )(p, eps, x, w_gat, att_src, att_dst, b_gat)


def _gcn_deg_body(p_ref, eps_ref, o_ref):
    i = pl.program_id(0)

    @pl.when(i == 0)
    def _():
        o_ref[...] = jnp.zeros_like(o_ref)

    eps = eps_ref[0]
    rows = lax.broadcasted_iota(jnp.int32, (_RB, N), 0) + i * _RB
    cols = lax.broadcasted_iota(jnp.int32, (_RB, N), 1)
    adjf = (p_ref[...] >= eps).astype(jnp.float32)
    ag = jnp.where(rows == cols, 1.0, adjf)
    o_ref[...] += jnp.sum(ag, axis=0, keepdims=True)


def _gcn_deg(p, eps):
    return pl.pallas_call(
        _gcn_deg_body,
        grid=(N // _RB,),
        in_specs=[
            pl.BlockSpec((_RB, N), lambda i: (i, 0)),
            pl.BlockSpec(memory_space=pltpu.SMEM),
        ],
        out_specs=pl.BlockSpec((1, N), lambda i: (0, 0)),
        out_shape=jax.ShapeDtypeStruct((1, N), jnp.float32),
    )(p, eps)


def _gcn_body(p_ref, eps_ref, dgt_ref, x1_ref, wg_ref, bg_ref, x2_ref):
    i = pl.program_id(0)
    eps = eps_ref[0]
    rows = lax.broadcasted_iota(jnp.int32, (N, _RB), 0)
    cols = lax.broadcasted_iota(jnp.int32, (N, _RB), 1) + i * _RB
    adjf = (p_ref[...] >= eps).astype(jnp.float32)
    ag = jnp.where(rows == cols, 1.0, adjf)
    dgi = lax.rsqrt(dgt_ref[...])
    y = jnp.dot(x1_ref[...], wg_ref[...],
                preferred_element_type=jnp.float32)
    w = dgi * y
    acc = lax.dot_general(ag, w, dimension_numbers=(((0,), (0,)), ((), ())),
                          preferred_element_type=jnp.float32)
    dgi_i = lax.rsqrt(dgt_ref[pl.ds(i * _RB, _RB), :])
    v = dgi_i * acc + bg_ref[...]
    x2_ref[...] = jnp.where(v > 0, v, jnp.exp(v) - 1.0)


def _gcn(p, eps, dg_t, x1, w_gcn, b_gcn):
    return pl.pallas_call(
        _gcn_body,
        grid=(N // _RB,),
        in_specs=[
            pl.BlockSpec((N, _RB), lambda i: (0, i)),
            pl.BlockSpec(memory_space=pltpu.SMEM),
            pl.BlockSpec((N, 1), lambda i: (0, 0)),
            pl.BlockSpec((N, 16), lambda i: (0, 0)),
            pl.BlockSpec((16, 32), lambda i: (0, 0)),
            pl.BlockSpec((1, 32), lambda i: (0, 0)),
        ],
        out_specs=pl.BlockSpec((_RB, 32), lambda i: (i, 0)),
        out_shape=jax.ShapeDtypeStruct((N, 32), jnp.float32),
    )(p, eps, dg_t, x1, w_gcn, b_gcn)


def _pool_body(x2_ref, b_ref, wl_ref, bl_ref, o_ref):
    x2 = x2_ref[...]
    batch = b_ref[...]
    giota = lax.broadcasted_iota(jnp.int32, (N, G), 1)
    segf = (batch == giota).astype(jnp.float32)
    ssum = lax.dot_general(segf, x2,
                           dimension_numbers=(((0,), (0,)), ((), ())),
                           preferred_element_type=jnp.float32)
    ones = jnp.ones((N, 1), jnp.float32)
    cnt = lax.dot_general(segf, ones,
                          dimension_numbers=(((0,), (0,)), ((), ())),
                          preferred_element_type=jnp.float32)
    smean = ssum / jnp.maximum(cnt, 1.0)
    rows = []
    for g in range(G):
        mg = jnp.max(jnp.where(batch == g, x2, -jnp.inf), axis=0,
                     keepdims=True)
        rows.append(mg)
    smax = jnp.concatenate(rows, axis=0)
    wl = wl_ref[...]
    out = (jnp.dot(smax, wl[0:32, :], preferred_element_type=jnp.float32)
           + jnp.dot(smean, wl[32:64, :], preferred_element_type=jnp.float32)
           + jnp.dot(ssum, wl[64:96, :], preferred_element_type=jnp.float32)
           + bl_ref[...])
    o_ref[...] = out


def _pool(x2, batch2d, w_lin, b_lin):
    return pl.pallas_call(
        _pool_body,
        grid=(1,),
        in_specs=[
            pl.BlockSpec((N, 32), lambda i: (0, 0)),
            pl.BlockSpec((N, 1), lambda i: (0, 0)),
            pl.BlockSpec((96, 2), lambda i: (0, 0)),
            pl.BlockSpec((1, 2), lambda i: (0, 0)),
        ],
        out_specs=pl.BlockSpec((G, 2), lambda i: (0, 0)),
        out_shape=jax.ShapeDtypeStruct((G, 2), jnp.float32),
    )(x2, batch2d, w_lin, b_lin)


def kernel(x, edge_index, batch, W_gat, att_src, att_dst, b_gat, W_gcn,
           b_gcn, W_lin, b_lin):
    src, dst = edge_index[0], edge_index[1]
    zero_blk = jnp.zeros((_ZSLICE * 16,), jnp.float32)
    a = _sc_scatter(src, dst, zero_blk).reshape(N, N)

    deg = _colsum(a) + 1.0
    deg_t = deg.reshape(N, 1)
    b, p = _normalize(a, deg, deg_t)

    bc = _mm(b, b)
    for j in range(1, 6):
        p = _mma(bc, p, p)
        if j < 5:
            bc = _mm(bc, bc)

    eps, dg = _select_eps(p)

    x1 = _gat(p, eps, x, W_gat,
              att_src.reshape(16, 1), att_dst.reshape(16, 1),
              b_gat.reshape(1, 16))
    x2 = _gcn(p, eps, dg.reshape(N, 1), x1, W_gcn, b_gcn.reshape(1, 32))
    out = _pool(x2, batch.reshape(N, 1), W_lin, b_lin.reshape(1, 2))
    return out
